# baseline (speedup 1.0000x reference)
import functools
import jax
import jax.numpy as jnp
from jax import lax
from jax.experimental import pallas as pl
from jax.experimental.pallas import tpu as pltpu
from jax.experimental.pallas import tpu_sc as plsc

D = 1024
E = 8
H = 512
N = 4096
NP = 2 * N
TM = 256
P_CAP = NP + E * TM
NTILES = P_CAP // TM
LANES = 128

NC, NS = 2, 16
NW = NC * NS


def _silu(x):
    return x / (1.0 + jnp.exp(-x))


def _router_body(x_ref, rw_ref, eb_ref, idx_ref, val_ref):
    x = x_ref[...]
    logits = jnp.dot(x, rw_ref[...], preferred_element_type=jnp.float32)
    logits = logits + eb_ref[...]
    sp = jnp.maximum(logits, 0.0) + jnp.log(1.0 + jnp.exp(-jnp.abs(logits)))
    act = jnp.sqrt(sp)
    m = x.shape[0]
    lane = lax.broadcasted_iota(jnp.int32, (m, LANES), 1)
    actm = jnp.where(lane < E, act, -1.0)
    m1 = jnp.max(actm, axis=1, keepdims=True)
    l1 = jnp.min(jnp.where(actm == m1, lane, LANES), axis=1, keepdims=True)
    act2 = jnp.where(lane == l1, -1.0, actm)
    m2 = jnp.max(act2, axis=1, keepdims=True)
    l2 = jnp.min(jnp.where(act2 == m2, lane, LANES), axis=1, keepdims=True)
    idx_ref[...] = jnp.concatenate([l1, l2], axis=1)
    val_ref[...] = jnp.concatenate([m1, m2], axis=1)


def _router(x2, routing_W, expert_bias):
    rw_pad = jnp.pad(routing_W, ((0, 0), (0, LANES - E)))
    eb_pad = jnp.pad(expert_bias, (0, LANES - E)).reshape(1, LANES)
    tm = 1024
    return pl.pallas_call(
        _router_body,
        grid=(N // tm,),
        in_specs=[
            pl.BlockSpec((tm, D), lambda t: (t, 0)),
            pl.BlockSpec((D, LANES), lambda t: (0, 0)),
            pl.BlockSpec((1, LANES), lambda t: (0, 0)),
        ],
        out_specs=[
            pl.BlockSpec((tm, 2), lambda t: (t, 0)),
            pl.BlockSpec((tm, 2), lambda t: (t, 0)),
        ],
        out_shape=[
            jax.ShapeDtypeStruct((N, 2), jnp.int32),
            jax.ShapeDtypeStruct((N, 2), jnp.float32),
        ],
    )(x2, rw_pad, eb_pad)


def _dispatch(top_idx, top_vals):
    eid = top_idx.reshape(-1)
    order = jnp.argsort(eid, stable=True)
    cnt = jnp.sum(eid[:, None] == jnp.arange(E)[None, :], axis=0,
                  dtype=jnp.int32)
    csum = jnp.cumsum(cnt)
    cstart = jnp.concatenate([jnp.zeros(1, jnp.int32), csum[:-1]])
    padded = ((cnt + TM - 1) // TM) * TM
    pstart = jnp.concatenate([jnp.zeros(1, jnp.int32), jnp.cumsum(padded)])
    e_of_r = eid[order]
    r = jnp.arange(NP, dtype=jnp.int32)
    p_of_r = pstart[e_of_r] + (r - cstart[e_of_r])
    pos = jnp.zeros((NP,), jnp.int32).at[order].set(p_of_r)
    gidx = jnp.zeros((P_CAP,), jnp.int32).at[p_of_r].set(order // 2)
    wgt = jnp.zeros((P_CAP,), jnp.float32).at[p_of_r].set(
        top_vals.reshape(-1)[order]).reshape(P_CAP, 1)
    tstart = jnp.arange(NTILES, dtype=jnp.int32) * TM
    te = jnp.sum(tstart[:, None] >= pstart[1:][None, :], axis=1,
                 dtype=jnp.int32)
    te = jnp.minimum(te, E - 1)
    return gidx, wgt, te, pos[0::2], pos[1::2]


RPW = P_CAP // NW
CH = 64


def _k3_body(x_hbm, gidx_hbm, xs_hbm, idx_c, rows_v, sem):
    wid = lax.axis_index("s") * NC + lax.axis_index("c")
    base = wid * RPW

    def chunk(c, _):
        off = base + c * CH
        pltpu.sync_copy(gidx_hbm.at[pl.ds(off, CH)], idx_c)
        pltpu.async_copy(x_hbm.at[idx_c], rows_v, sem).wait()
        pltpu.sync_copy(rows_v, xs_hbm.at[pl.ds(off, CH)])
        return 0

    lax.fori_loop(0, RPW // CH, chunk, 0)


def _sc_gather(x2, gidx):
    mesh = plsc.VectorSubcoreMesh(core_axis_name="c", subcore_axis_name="s")
    f = functools.partial(
        pl.kernel, mesh=mesh,
        out_type=jax.ShapeDtypeStruct((P_CAP, D), jnp.float32),
        scratch_types=[
            pltpu.VMEM((CH,), jnp.int32),
            pltpu.VMEM((CH, D), jnp.float32),
            pltpu.SemaphoreType.DMA,
        ],
    )(_k3_body)
    return f(x2, gidx)


def _group_body(te_ref, xs_ref, w1_ref, b1_ref, w2_ref, b2_ref, wgt_ref,
                y_ref):
    x = xs_ref[...]
    h = jnp.dot(x, w1_ref[0], preferred_element_type=jnp.float32) + b1_ref[0]
    h = _silu(h)
    y = jnp.dot(h, w2_ref[0], preferred_element_type=jnp.float32) + b2_ref[0]
    y_ref[...] = y * wgt_ref[...]


def _grouped_ffn(xs, routed_W1, routed_b1, routed_W2, routed_b2, wgt, te):
    b1 = routed_b1.reshape(E, 1, H)
    b2 = routed_b2.reshape(E, 1, D)
    grid_spec = pltpu.PrefetchScalarGridSpec(
        num_scalar_prefetch=1,
        grid=(NTILES,),
        in_specs=[
            pl.BlockSpec((TM, D), lambda t, te: (t, 0)),
            pl.BlockSpec((1, D, H), lambda t, te: (te[t], 0, 0)),
            pl.BlockSpec((1, 1, H), lambda t, te: (te[t], 0, 0)),
            pl.BlockSpec((1, H, D), lambda t, te: (te[t], 0, 0)),
            pl.BlockSpec((1, 1, D), lambda t, te: (te[t], 0, 0)),
            pl.BlockSpec((TM, 1), lambda t, te: (t, 0)),
        ],
        out_specs=pl.BlockSpec((TM, D), lambda t, te: (t, 0)),
    )
    return pl.pallas_call(
        _group_body,
        grid_spec=grid_spec,
        out_shape=jax.ShapeDtypeStruct((P_CAP, D), jnp.float32),
        compiler_params=pltpu.CompilerParams(
            dimension_semantics=("arbitrary",),
        ),
    )(te, xs, routed_W1, b1, routed_W2, b2, wgt)


def _shared_body(x_ref, w1_ref, b1_ref, w2_ref, b2_ref, s_ref):
    x = x_ref[...]
    h = jnp.dot(x, w1_ref[...], preferred_element_type=jnp.float32) + b1_ref[...]
    h = _silu(h)
    s_ref[...] = jnp.dot(h, w2_ref[...], preferred_element_type=jnp.float32) \
        + b2_ref[...]


def _shared_ffn(x2, W1, b1, W2, b2):
    tm = 512
    return pl.pallas_call(
        _shared_body,
        grid=(N // tm,),
        in_specs=[
            pl.BlockSpec((tm, D), lambda t: (t, 0)),
            pl.BlockSpec((D, H), lambda t: (0, 0)),
            pl.BlockSpec((1, H), lambda t: (0, 0)),
            pl.BlockSpec((H, D), lambda t: (0, 0)),
            pl.BlockSpec((1, D), lambda t: (0, 0)),
        ],
        out_specs=pl.BlockSpec((tm, D), lambda t: (t, 0)),
        out_shape=jax.ShapeDtypeStruct((N, D), jnp.float32),
    )(x2, W1, b1.reshape(1, H), W2, b2.reshape(1, D))


TPW = N // NW
CC = 16


def _k6_body(s_hbm, y_hbm, pos0_hbm, pos1_hbm, out_hbm,
             p0_v, p1_v, y0_v, y1_v, s_v, sem):
    wid = lax.axis_index("s") * NC + lax.axis_index("c")
    base = wid * TPW

    def chunk(c, _):
        off = base + c * CC
        pltpu.sync_copy(pos0_hbm.at[pl.ds(off, CC)], p0_v)
        pltpu.sync_copy(pos1_hbm.at[pl.ds(off, CC)], p1_v)
        pltpu.async_copy(y_hbm.at[p0_v], y0_v, sem).wait()
        pltpu.async_copy(y_hbm.at[p1_v], y1_v, sem).wait()
        pltpu.sync_copy(s_hbm.at[pl.ds(off, CC)], s_v)
        for r in range(CC):
            def col(g, _):
                sl = pl.ds(g * 16, 16)
                s_v[r, sl] = s_v[r, sl] + y0_v[r, sl] + y1_v[r, sl]
                return 0
            lax.fori_loop(0, D // 16, col, 0)
        pltpu.sync_copy(s_v, out_hbm.at[pl.ds(off, CC)])
        return 0

    lax.fori_loop(0, TPW // CC, chunk, 0)


def _sc_combine(S, Y, pos0, pos1):
    mesh = plsc.VectorSubcoreMesh(core_axis_name="c", subcore_axis_name="s")
    f = functools.partial(
        pl.kernel, mesh=mesh,
        out_type=jax.ShapeDtypeStruct((N, D), jnp.float32),
        scratch_types=[
            pltpu.VMEM((CC,), jnp.int32),
            pltpu.VMEM((CC,), jnp.int32),
            pltpu.VMEM((CC, D), jnp.float32),
            pltpu.VMEM((CC, D), jnp.float32),
            pltpu.VMEM((CC, D), jnp.float32),
            pltpu.SemaphoreType.DMA,
        ],
    )(_k6_body)
    return f(S, Y, pos0, pos1)


def kernel(X, shared_W1, shared_b1, shared_W2, shared_b2,
           routed_W1, routed_b1, routed_W2, routed_b2,
           routing_W, expert_bias):
    B, T, _ = X.shape
    x2 = X.reshape(N, D)

    top_idx, top_vals = _router(x2, routing_W, expert_bias)
    gidx, wgt, te, pos0, pos1 = _dispatch(top_idx, top_vals)
    xs = _sc_gather(x2, gidx)
    y = _grouped_ffn(xs, routed_W1, routed_b1, routed_W2, routed_b2, wgt, te)
    s = _shared_ffn(x2, shared_W1, shared_b1, shared_W2, shared_b2)
    out = _sc_combine(s, y, pos0, pos1)
    return out.reshape(B, T, D)


# speedup vs baseline: 1.0823x; 1.0823x over previous
import functools
import jax
import jax.numpy as jnp
from jax import lax
from jax.experimental import pallas as pl
from jax.experimental.pallas import tpu as pltpu
from jax.experimental.pallas import tpu_sc as plsc

D = 1024
E = 8
H = 512
N = 4096
NP = 2 * N
TM = 256
P_CAP = NP + E * TM
NTILES = P_CAP // TM
LANES = 128

NC, NS = 2, 16
NW = NC * NS


def _silu(x):
    return x / (1.0 + jnp.exp(-x))


def _router_body(x_ref, rw_ref, eb_ref, idx_ref, val_ref):
    x = x_ref[...]
    logits = jnp.dot(x, rw_ref[...], preferred_element_type=jnp.float32)
    logits = logits + eb_ref[...]
    sp = jnp.maximum(logits, 0.0) + jnp.log(1.0 + jnp.exp(-jnp.abs(logits)))
    act = jnp.sqrt(sp)
    m = x.shape[0]
    lane = lax.broadcasted_iota(jnp.int32, (m, LANES), 1)
    actm = jnp.where(lane < E, act, -1.0)
    m1 = jnp.max(actm, axis=1, keepdims=True)
    l1 = jnp.min(jnp.where(actm == m1, lane, LANES), axis=1, keepdims=True)
    act2 = jnp.where(lane == l1, -1.0, actm)
    m2 = jnp.max(act2, axis=1, keepdims=True)
    l2 = jnp.min(jnp.where(act2 == m2, lane, LANES), axis=1, keepdims=True)
    idx_ref[...] = jnp.concatenate([l1, l2], axis=1)
    val_ref[...] = jnp.concatenate([m1, m2], axis=1)


def _router(x2, routing_W, expert_bias):
    rw_pad = jnp.pad(routing_W, ((0, 0), (0, LANES - E)))
    eb_pad = jnp.pad(expert_bias, (0, LANES - E)).reshape(1, LANES)
    tm = 1024
    return pl.pallas_call(
        _router_body,
        grid=(N // tm,),
        in_specs=[
            pl.BlockSpec((tm, D), lambda t: (t, 0)),
            pl.BlockSpec((D, LANES), lambda t: (0, 0)),
            pl.BlockSpec((1, LANES), lambda t: (0, 0)),
        ],
        out_specs=[
            pl.BlockSpec((tm, 2), lambda t: (t, 0)),
            pl.BlockSpec((tm, 2), lambda t: (t, 0)),
        ],
        out_shape=[
            jax.ShapeDtypeStruct((N, 2), jnp.int32),
            jax.ShapeDtypeStruct((N, 2), jnp.float32),
        ],
    )(x2, rw_pad, eb_pad)


def _dispatch(top_idx, top_vals):
    eid = top_idx.reshape(-1)
    order = jnp.argsort(eid, stable=True)
    cnt = jnp.sum(eid[:, None] == jnp.arange(E)[None, :], axis=0,
                  dtype=jnp.int32)
    csum = jnp.cumsum(cnt)
    cstart = jnp.concatenate([jnp.zeros(1, jnp.int32), csum[:-1]])
    padded = ((cnt + TM - 1) // TM) * TM
    pstart = jnp.concatenate([jnp.zeros(1, jnp.int32), jnp.cumsum(padded)])
    e_of_r = eid[order]
    r = jnp.arange(NP, dtype=jnp.int32)
    p_of_r = pstart[e_of_r] + (r - cstart[e_of_r])
    pos = jnp.zeros((NP,), jnp.int32).at[order].set(p_of_r)
    gidx = jnp.zeros((P_CAP,), jnp.int32).at[p_of_r].set(order // 2)
    wgt = jnp.zeros((P_CAP,), jnp.float32).at[p_of_r].set(
        top_vals.reshape(-1)[order]).reshape(P_CAP, 1)
    tstart = jnp.arange(NTILES, dtype=jnp.int32) * TM
    te = jnp.sum(tstart[:, None] >= pstart[1:][None, :], axis=1,
                 dtype=jnp.int32)
    te = jnp.minimum(te, E - 1)
    return gidx, wgt, te, pos


RPW = P_CAP // NW
CH = 40
NCH = RPW // CH


def _k3_body(x_hbm, gidx_hbm, xs_hbm,
             idx_b0, idx_b1, rows0, rows1,
             sem_i0, sem_i1, sem_g0, sem_g1, sem_w0, sem_w1):
    wid = lax.axis_index("s") * NC + lax.axis_index("c")
    base = wid * RPW
    idx_b = [idx_b0, idx_b1]
    rows = [rows0, rows1]
    sem_i = [sem_i0, sem_i1]
    sem_g = [sem_g0, sem_g1]
    sem_w = [sem_w0, sem_w1]

    def load_idx(c):
        b = c % 2
        return pltpu.async_copy(gidx_hbm.at[pl.ds(base + c * CH, CH)],
                                idx_b[b], sem_i[b])

    gd = [None] * NCH
    wd = [None] * NCH
    id0 = load_idx(0)
    id1 = load_idx(1)
    idn = [id0, id1]
    idn[0].wait()
    gd[0] = pltpu.async_copy(x_hbm.at[idx_b[0]], rows[0], sem_g[0])
    for c in range(NCH):
        if c + 1 < NCH:
            nb = (c + 1) % 2
            idn[nb].wait()
            if c >= 1:
                wd[c - 1].wait()
            gd[c + 1] = pltpu.async_copy(x_hbm.at[idx_b[nb]], rows[nb],
                                         sem_g[nb])
        gd[c].wait()
        if c + 2 < NCH:
            idn[c % 2] = load_idx(c + 2)
        wd[c] = pltpu.async_copy(rows[c % 2],
                                 xs_hbm.at[pl.ds(base + c * CH, CH)],
                                 sem_w[c % 2])
    wd[NCH - 2].wait()
    wd[NCH - 1].wait()


def _sc_gather(x2, gidx):
    mesh = plsc.VectorSubcoreMesh(core_axis_name="c", subcore_axis_name="s")
    f = functools.partial(
        pl.kernel, mesh=mesh,
        out_type=jax.ShapeDtypeStruct((P_CAP, D), jnp.float32),
        scratch_types=[
            pltpu.VMEM((CH,), jnp.int32),
            pltpu.VMEM((CH,), jnp.int32),
            pltpu.VMEM((CH, D), jnp.float32),
            pltpu.VMEM((CH, D), jnp.float32),
            pltpu.SemaphoreType.DMA,
            pltpu.SemaphoreType.DMA,
            pltpu.SemaphoreType.DMA,
            pltpu.SemaphoreType.DMA,
            pltpu.SemaphoreType.DMA,
            pltpu.SemaphoreType.DMA,
        ],
    )(_k3_body)
    return f(x2, gidx)


def _group_body(te_ref, xs_ref, w1_ref, b1_ref, w2_ref, b2_ref, wgt_ref,
                y_ref):
    x = xs_ref[...]
    h = jnp.dot(x, w1_ref[0], preferred_element_type=jnp.float32) + b1_ref[0]
    h = _silu(h)
    y = jnp.dot(h, w2_ref[0], preferred_element_type=jnp.float32) + b2_ref[0]
    y_ref[...] = y * wgt_ref[...]


def _grouped_ffn(xs, routed_W1, routed_b1, routed_W2, routed_b2, wgt, te):
    b1 = routed_b1.reshape(E, 1, H)
    b2 = routed_b2.reshape(E, 1, D)
    grid_spec = pltpu.PrefetchScalarGridSpec(
        num_scalar_prefetch=1,
        grid=(NTILES,),
        in_specs=[
            pl.BlockSpec((TM, D), lambda t, te: (t, 0)),
            pl.BlockSpec((1, D, H), lambda t, te: (te[t], 0, 0)),
            pl.BlockSpec((1, 1, H), lambda t, te: (te[t], 0, 0)),
            pl.BlockSpec((1, H, D), lambda t, te: (te[t], 0, 0)),
            pl.BlockSpec((1, 1, D), lambda t, te: (te[t], 0, 0)),
            pl.BlockSpec((TM, 1), lambda t, te: (t, 0)),
        ],
        out_specs=pl.BlockSpec((TM, D), lambda t, te: (t, 0)),
    )
    return pl.pallas_call(
        _group_body,
        grid_spec=grid_spec,
        out_shape=jax.ShapeDtypeStruct((P_CAP, D), jnp.float32),
        compiler_params=pltpu.CompilerParams(
            dimension_semantics=("arbitrary",),
        ),
    )(te, xs, routed_W1, b1, routed_W2, b2, wgt)


def _shared_body(x_ref, w1_ref, b1_ref, w2_ref, b2_ref, s_ref):
    x = x_ref[...]
    h = jnp.dot(x, w1_ref[...], preferred_element_type=jnp.float32) + b1_ref[...]
    h = _silu(h)
    s_ref[...] = jnp.dot(h, w2_ref[...], preferred_element_type=jnp.float32) \
        + b2_ref[...]


def _shared_ffn(x2, W1, b1, W2, b2):
    tm = 512
    return pl.pallas_call(
        _shared_body,
        grid=(N // tm,),
        in_specs=[
            pl.BlockSpec((tm, D), lambda t: (t, 0)),
            pl.BlockSpec((D, H), lambda t: (0, 0)),
            pl.BlockSpec((1, H), lambda t: (0, 0)),
            pl.BlockSpec((H, D), lambda t: (0, 0)),
            pl.BlockSpec((1, D), lambda t: (0, 0)),
        ],
        out_specs=pl.BlockSpec((tm, D), lambda t: (t, 0)),
        out_shape=jax.ShapeDtypeStruct((N, D), jnp.float32),
    )(x2, W1, b1.reshape(1, H), W2, b2.reshape(1, D))


TPW = N // NW
CC = 16
NCC = TPW // CC


def _k6_body(s_hbm, y_hbm, pos_hbm, out_hbm,
             pidx_b0, pidx_b1, y01_0, y01_1, s_0, s_1,
             sem_i0, sem_i1, sem_g0, sem_g1, sem_s0, sem_s1,
             sem_w0, sem_w1):
    wid = lax.axis_index("s") * NC + lax.axis_index("c")
    base = wid * TPW
    pidx_b = [pidx_b0, pidx_b1]
    y01 = [y01_0, y01_1]
    sb = [s_0, s_1]
    sem_i = [sem_i0, sem_i1]
    sem_g = [sem_g0, sem_g1]
    sem_s = [sem_s0, sem_s1]
    sem_w = [sem_w0, sem_w1]

    def load_pidx(c):
        b = c % 2
        return pltpu.async_copy(pos_hbm.at[pl.ds(2 * (base + c * CC), 2 * CC)],
                                pidx_b[b], sem_i[b])

    gd = [None] * NCC
    sd = [None] * NCC
    wd = [None] * NCC
    idn = [load_pidx(0), load_pidx(1)]
    idn[0].wait()
    gd[0] = pltpu.async_copy(y_hbm.at[pidx_b[0]], y01[0], sem_g[0])
    sd[0] = pltpu.async_copy(s_hbm.at[pl.ds(base, CC)], sb[0], sem_s[0])
    for c in range(NCC):
        b = c % 2
        if c + 1 < NCC:
            nb = (c + 1) % 2
            idn[nb].wait()
            if c >= 1:
                wd[c - 1].wait()
            gd[c + 1] = pltpu.async_copy(y_hbm.at[pidx_b[nb]], y01[nb],
                                         sem_g[nb])
            sd[c + 1] = pltpu.async_copy(
                s_hbm.at[pl.ds(base + (c + 1) * CC, CC)], sb[nb], sem_s[nb])
        gd[c].wait()
        if c + 2 < NCC:
            idn[b] = load_pidx(c + 2)
        sd[c].wait()

        def row(r, _):
            def col(i, _):
                for u in range(4):
                    cs = pl.ds((i * 4 + u) * 16, 16)
                    sb[b][r, cs] = (sb[b][r, cs] + y01[b][2 * r, cs]
                                    + y01[b][2 * r + 1, cs])
                return 0
            lax.fori_loop(0, D // 64, col, 0)
            return 0

        lax.fori_loop(0, CC, row, 0)
        wd[c] = pltpu.async_copy(sb[b], out_hbm.at[pl.ds(base + c * CC, CC)],
                                 sem_w[b])
    wd[NCC - 2].wait()
    wd[NCC - 1].wait()


def _sc_combine(S, Y, pos):
    mesh = plsc.VectorSubcoreMesh(core_axis_name="c", subcore_axis_name="s")
    f = functools.partial(
        pl.kernel, mesh=mesh,
        out_type=jax.ShapeDtypeStruct((N, D), jnp.float32),
        scratch_types=[
            pltpu.VMEM((2 * CC,), jnp.int32),
            pltpu.VMEM((2 * CC,), jnp.int32),
            pltpu.VMEM((2 * CC, D), jnp.float32),
            pltpu.VMEM((2 * CC, D), jnp.float32),
            pltpu.VMEM((CC, D), jnp.float32),
            pltpu.VMEM((CC, D), jnp.float32),
            pltpu.SemaphoreType.DMA,
            pltpu.SemaphoreType.DMA,
            pltpu.SemaphoreType.DMA,
            pltpu.SemaphoreType.DMA,
            pltpu.SemaphoreType.DMA,
            pltpu.SemaphoreType.DMA,
            pltpu.SemaphoreType.DMA,
            pltpu.SemaphoreType.DMA,
        ],
    )(_k6_body)
    return f(S, Y, pos)


def kernel(X, shared_W1, shared_b1, shared_W2, shared_b2,
           routed_W1, routed_b1, routed_W2, routed_b2,
           routing_W, expert_bias):
    B, T, _ = X.shape
    x2 = X.reshape(N, D)

    top_idx, top_vals = _router(x2, routing_W, expert_bias)
    gidx, wgt, te, pos = _dispatch(top_idx, top_vals)
    xs = _sc_gather(x2, gidx)
    y = _grouped_ffn(xs, routed_W1, routed_b1, routed_W2, routed_b2, wgt, te)
    s = _shared_ffn(x2, shared_W1, shared_b1, shared_W2, shared_b2)
    out = _sc_combine(s, y, pos)
    return out.reshape(B, T, D)


# speedup vs baseline: 1.1297x; 1.0438x over previous
import functools
import jax
import jax.numpy as jnp
from jax import lax
from jax.experimental import pallas as pl
from jax.experimental.pallas import tpu as pltpu
from jax.experimental.pallas import tpu_sc as plsc

D = 1024
E = 8
H = 512
N = 4096
NP = 2 * N
TM = 256
P_CAP = NP + E * TM
NTILES = P_CAP // TM
LANES = 128

NC, NS = 2, 16
NW = NC * NS


def _silu(x):
    return x / (1.0 + jnp.exp(-x))


def _pack_bf16_pair(a, b):
    au = jax.lax.bitcast_convert_type(a, jnp.uint32) >> 16
    bu = jax.lax.bitcast_convert_type(b, jnp.uint32) & jnp.uint32(0xFFFF0000)
    return jax.lax.bitcast_convert_type(au | bu, jnp.int32)


def _unpack_bf16_pair(xi):
    xu = jax.lax.bitcast_convert_type(xi, jnp.uint32)
    lo = jax.lax.bitcast_convert_type(xu << 16, jnp.float32)
    hi = jax.lax.bitcast_convert_type(xu & jnp.uint32(0xFFFF0000), jnp.float32)
    return lo, hi


def _router_body(x_ref, rw_ref, eb_ref, idx_ref, val_ref, xp_ref):
    x = x_ref[...]
    xp_ref[...] = _pack_bf16_pair(x[:, :D // 2], x[:, D // 2:])
    logits = jnp.dot(x, rw_ref[...], preferred_element_type=jnp.float32)
    logits = logits + eb_ref[...]
    sp = jnp.maximum(logits, 0.0) + jnp.log(1.0 + jnp.exp(-jnp.abs(logits)))
    act = jnp.sqrt(sp)
    m = x.shape[0]
    lane = lax.broadcasted_iota(jnp.int32, (m, LANES), 1)
    actm = jnp.where(lane < E, act, -1.0)
    m1 = jnp.max(actm, axis=1, keepdims=True)
    l1 = jnp.min(jnp.where(actm == m1, lane, LANES), axis=1, keepdims=True)
    act2 = jnp.where(lane == l1, -1.0, actm)
    m2 = jnp.max(act2, axis=1, keepdims=True)
    l2 = jnp.min(jnp.where(act2 == m2, lane, LANES), axis=1, keepdims=True)
    idx_ref[...] = jnp.concatenate([l1, l2], axis=1)
    val_ref[...] = jnp.concatenate([m1, m2], axis=1)


def _router(x2, routing_W, expert_bias):
    rw_pad = jnp.pad(routing_W, ((0, 0), (0, LANES - E)))
    eb_pad = jnp.pad(expert_bias, (0, LANES - E)).reshape(1, LANES)
    tm = 1024
    return pl.pallas_call(
        _router_body,
        grid=(N // tm,),
        in_specs=[
            pl.BlockSpec((tm, D), lambda t: (t, 0)),
            pl.BlockSpec((D, LANES), lambda t: (0, 0)),
            pl.BlockSpec((1, LANES), lambda t: (0, 0)),
        ],
        out_specs=[
            pl.BlockSpec((tm, 2), lambda t: (t, 0)),
            pl.BlockSpec((tm, 2), lambda t: (t, 0)),
            pl.BlockSpec((tm, D // 2), lambda t: (t, 0)),
        ],
        out_shape=[
            jax.ShapeDtypeStruct((N, 2), jnp.int32),
            jax.ShapeDtypeStruct((N, 2), jnp.float32),
            jax.ShapeDtypeStruct((N, D // 2), jnp.int32),
        ],
    )(x2, rw_pad, eb_pad)


def _dispatch(top_idx, top_vals):
    eid = top_idx.reshape(-1)
    order = jnp.argsort(eid, stable=True)
    cnt = jnp.sum(eid[:, None] == jnp.arange(E)[None, :], axis=0,
                  dtype=jnp.int32)
    csum = jnp.cumsum(cnt)
    cstart = jnp.concatenate([jnp.zeros(1, jnp.int32), csum[:-1]])
    padded = ((cnt + TM - 1) // TM) * TM
    pstart = jnp.concatenate([jnp.zeros(1, jnp.int32), jnp.cumsum(padded)])
    e_of_r = eid[order]
    r = jnp.arange(NP, dtype=jnp.int32)
    p_of_r = pstart[e_of_r] + (r - cstart[e_of_r])
    pos = jnp.zeros((NP,), jnp.int32).at[order].set(p_of_r)
    gidx = jnp.zeros((P_CAP,), jnp.int32).at[p_of_r].set(order // 2)
    wgt = jnp.zeros((P_CAP,), jnp.float32).at[p_of_r].set(
        top_vals.reshape(-1)[order]).reshape(P_CAP, 1)
    tstart = jnp.arange(NTILES, dtype=jnp.int32) * TM
    te = jnp.sum(tstart[:, None] >= pstart[1:][None, :], axis=1,
                 dtype=jnp.int32)
    te = jnp.minimum(te, E - 1)
    return gidx, wgt, te, pos


RPW = P_CAP // NW
CH = 64
NCH = RPW // CH


def _k3_body(x_hbm, gidx_hbm, xs_hbm,
             idx_b0, idx_b1, rows0, rows1,
             sem_i0, sem_i1, sem_g0, sem_g1, sem_w0, sem_w1):
    wid = lax.axis_index("s") * NC + lax.axis_index("c")
    base = wid * RPW
    idx_b = [idx_b0, idx_b1]
    rows = [rows0, rows1]
    sem_i = [sem_i0, sem_i1]
    sem_g = [sem_g0, sem_g1]
    sem_w = [sem_w0, sem_w1]

    def load_idx(c):
        b = c % 2
        return pltpu.async_copy(gidx_hbm.at[pl.ds(base + c * CH, CH)],
                                idx_b[b], sem_i[b])

    gd = [None] * NCH
    wd = [None] * NCH
    id0 = load_idx(0)
    id1 = load_idx(1)
    idn = [id0, id1]
    idn[0].wait()
    gd[0] = pltpu.async_copy(x_hbm.at[idx_b[0]], rows[0], sem_g[0])
    for c in range(NCH):
        if c + 1 < NCH:
            nb = (c + 1) % 2
            idn[nb].wait()
            if c >= 1:
                wd[c - 1].wait()
            gd[c + 1] = pltpu.async_copy(x_hbm.at[idx_b[nb]], rows[nb],
                                         sem_g[nb])
        gd[c].wait()
        if c + 2 < NCH:
            idn[c % 2] = load_idx(c + 2)
        wd[c] = pltpu.async_copy(rows[c % 2],
                                 xs_hbm.at[pl.ds(base + c * CH, CH)],
                                 sem_w[c % 2])
    wd[NCH - 2].wait()
    wd[NCH - 1].wait()


def _sc_gather(x2, gidx):
    mesh = plsc.VectorSubcoreMesh(core_axis_name="c", subcore_axis_name="s")
    f = functools.partial(
        pl.kernel, mesh=mesh,
        out_type=jax.ShapeDtypeStruct((P_CAP, D // 2), jnp.int32),
        scratch_types=[
            pltpu.VMEM((CH,), jnp.int32),
            pltpu.VMEM((CH,), jnp.int32),
            pltpu.VMEM((CH, D // 2), jnp.int32),
            pltpu.VMEM((CH, D // 2), jnp.int32),
            pltpu.SemaphoreType.DMA,
            pltpu.SemaphoreType.DMA,
            pltpu.SemaphoreType.DMA,
            pltpu.SemaphoreType.DMA,
            pltpu.SemaphoreType.DMA,
            pltpu.SemaphoreType.DMA,
        ],
    )(_k3_body)
    return f(x2, gidx)


def _group_body(te_ref, xs_ref, w1_ref, b1_ref, w2_ref, b2_ref, wgt_ref,
                y_ref):
    xlo, xhi = _unpack_bf16_pair(xs_ref[...])
    w1 = w1_ref[0]
    h = jnp.dot(xlo.astype(jnp.bfloat16),
                w1[:D // 2, :].astype(jnp.bfloat16),
                preferred_element_type=jnp.float32)
    h += jnp.dot(xhi.astype(jnp.bfloat16),
                 w1[D // 2:, :].astype(jnp.bfloat16),
                 preferred_element_type=jnp.float32)
    h += b1_ref[0]
    h = _silu(h)
    w2b = w2_ref[0].astype(jnp.bfloat16)
    y = jnp.dot(h.astype(jnp.bfloat16), w2b,
                preferred_element_type=jnp.float32) + b2_ref[0]
    y_ref[...] = y * wgt_ref[...]


def _grouped_ffn(xs, routed_W1, routed_b1, routed_W2, routed_b2, wgt, te):
    b1 = routed_b1.reshape(E, 1, H)
    b2 = routed_b2.reshape(E, 1, D)
    grid_spec = pltpu.PrefetchScalarGridSpec(
        num_scalar_prefetch=1,
        grid=(NTILES,),
        in_specs=[
            pl.BlockSpec((TM, D // 2), lambda t, te: (t, 0)),
            pl.BlockSpec((1, D, H), lambda t, te: (te[t], 0, 0)),
            pl.BlockSpec((1, 1, H), lambda t, te: (te[t], 0, 0)),
            pl.BlockSpec((1, H, D), lambda t, te: (te[t], 0, 0)),
            pl.BlockSpec((1, 1, D), lambda t, te: (te[t], 0, 0)),
            pl.BlockSpec((TM, 1), lambda t, te: (t, 0)),
        ],
        out_specs=pl.BlockSpec((TM, D), lambda t, te: (t, 0)),
    )
    return pl.pallas_call(
        _group_body,
        grid_spec=grid_spec,
        out_shape=jax.ShapeDtypeStruct((P_CAP, D), jnp.float32),
        compiler_params=pltpu.CompilerParams(
            dimension_semantics=("arbitrary",),
        ),
    )(te, xs, routed_W1, b1, routed_W2, b2, wgt)


def _shared_body(x_ref, w1_ref, b1_ref, w2_ref, b2_ref, s_ref):
    x = x_ref[...]
    h = jnp.dot(x, w1_ref[...], preferred_element_type=jnp.float32) + b1_ref[...]
    h = _silu(h)
    s_ref[...] = jnp.dot(h, w2_ref[...], preferred_element_type=jnp.float32) \
        + b2_ref[...]


def _shared_ffn(x2, W1, b1, W2, b2):
    tm = 512
    return pl.pallas_call(
        _shared_body,
        grid=(N // tm,),
        in_specs=[
            pl.BlockSpec((tm, D), lambda t: (t, 0)),
            pl.BlockSpec((D, H), lambda t: (0, 0)),
            pl.BlockSpec((1, H), lambda t: (0, 0)),
            pl.BlockSpec((H, D), lambda t: (0, 0)),
            pl.BlockSpec((1, D), lambda t: (0, 0)),
        ],
        out_specs=pl.BlockSpec((tm, D), lambda t: (t, 0)),
        out_shape=jax.ShapeDtypeStruct((N, D), jnp.float32),
    )(x2, W1, b1.reshape(1, H), W2, b2.reshape(1, D))


TPW = N // NW
CC = 16
NCC = TPW // CC


def _k6_body(s_hbm, y_hbm, pos_hbm, out_hbm,
             pidx_b0, pidx_b1, y01_0, y01_1, s_0, s_1,
             sem_i0, sem_i1, sem_g0, sem_g1, sem_s0, sem_s1,
             sem_w0, sem_w1):
    wid = lax.axis_index("s") * NC + lax.axis_index("c")
    base = wid * TPW
    pidx_b = [pidx_b0, pidx_b1]
    y01 = [y01_0, y01_1]
    sb = [s_0, s_1]
    sem_i = [sem_i0, sem_i1]
    sem_g = [sem_g0, sem_g1]
    sem_s = [sem_s0, sem_s1]
    sem_w = [sem_w0, sem_w1]

    def load_pidx(c):
        b = c % 2
        return pltpu.async_copy(pos_hbm.at[pl.ds(2 * (base + c * CC), 2 * CC)],
                                pidx_b[b], sem_i[b])

    gd = [None] * NCC
    sd = [None] * NCC
    wd = [None] * NCC
    idn = [load_pidx(0), load_pidx(1)]
    idn[0].wait()
    gd[0] = pltpu.async_copy(y_hbm.at[pidx_b[0]], y01[0], sem_g[0])
    sd[0] = pltpu.async_copy(s_hbm.at[pl.ds(base, CC)], sb[0], sem_s[0])
    for c in range(NCC):
        b = c % 2
        if c + 1 < NCC:
            nb = (c + 1) % 2
            idn[nb].wait()
            if c >= 1:
                wd[c - 1].wait()
            gd[c + 1] = pltpu.async_copy(y_hbm.at[pidx_b[nb]], y01[nb],
                                         sem_g[nb])
            sd[c + 1] = pltpu.async_copy(
                s_hbm.at[pl.ds(base + (c + 1) * CC, CC)], sb[nb], sem_s[nb])
        gd[c].wait()
        if c + 2 < NCC:
            idn[b] = load_pidx(c + 2)
        sd[c].wait()

        def row(r, _):
            def col(i, _):
                for u in range(4):
                    cs = pl.ds((i * 4 + u) * 16, 16)
                    sb[b][r, cs] = (sb[b][r, cs] + y01[b][2 * r, cs]
                                    + y01[b][2 * r + 1, cs])
                return 0
            lax.fori_loop(0, D // 64, col, 0)
            return 0

        lax.fori_loop(0, CC, row, 0)
        wd[c] = pltpu.async_copy(sb[b], out_hbm.at[pl.ds(base + c * CC, CC)],
                                 sem_w[b])
    wd[NCC - 2].wait()
    wd[NCC - 1].wait()


def _sc_combine(S, Y, pos):
    mesh = plsc.VectorSubcoreMesh(core_axis_name="c", subcore_axis_name="s")
    f = functools.partial(
        pl.kernel, mesh=mesh,
        out_type=jax.ShapeDtypeStruct((N, D), jnp.float32),
        scratch_types=[
            pltpu.VMEM((2 * CC,), jnp.int32),
            pltpu.VMEM((2 * CC,), jnp.int32),
            pltpu.VMEM((2 * CC, D), jnp.float32),
            pltpu.VMEM((2 * CC, D), jnp.float32),
            pltpu.VMEM((CC, D), jnp.float32),
            pltpu.VMEM((CC, D), jnp.float32),
            pltpu.SemaphoreType.DMA,
            pltpu.SemaphoreType.DMA,
            pltpu.SemaphoreType.DMA,
            pltpu.SemaphoreType.DMA,
            pltpu.SemaphoreType.DMA,
            pltpu.SemaphoreType.DMA,
            pltpu.SemaphoreType.DMA,
            pltpu.SemaphoreType.DMA,
        ],
    )(_k6_body)
    return f(S, Y, pos)


def kernel(X, shared_W1, shared_b1, shared_W2, shared_b2,
           routed_W1, routed_b1, routed_W2, routed_b2,
           routing_W, expert_bias):
    B, T, _ = X.shape
    x2 = X.reshape(N, D)

    top_idx, top_vals, xp = _router(x2, routing_W, expert_bias)
    gidx, wgt, te, pos = _dispatch(top_idx, top_vals)
    xs = _sc_gather(xp, gidx)
    y = _grouped_ffn(xs, routed_W1, routed_b1, routed_W2, routed_b2, wgt, te)
    s = _shared_ffn(x2, shared_W1, shared_b1, shared_W2, shared_b2)
    out = _sc_combine(s, y, pos)
    return out.reshape(B, T, D)


# speedup vs baseline: 1.1941x; 1.0570x over previous
import functools
import jax
import jax.numpy as jnp
from jax import lax
from jax.experimental import pallas as pl
from jax.experimental.pallas import tpu as pltpu
from jax.experimental.pallas import tpu_sc as plsc

D = 1024
E = 8
H = 512
N = 4096
NP = 2 * N
TM = 256
P_CAP = NP + E * TM
NTILES = P_CAP // TM
LANES = 128

NC, NS = 2, 16
NW = NC * NS


def _silu(x):
    return x / (1.0 + jnp.exp(-x))


def _pack_bf16_pair(a, b):
    au = jax.lax.bitcast_convert_type(a, jnp.uint32) >> 16
    bu = jax.lax.bitcast_convert_type(b, jnp.uint32) & jnp.uint32(0xFFFF0000)
    return jax.lax.bitcast_convert_type(au | bu, jnp.int32)


def _unpack_bf16_pair(xi):
    xu = jax.lax.bitcast_convert_type(xi, jnp.uint32)
    lo = jax.lax.bitcast_convert_type(xu << 16, jnp.float32)
    hi = jax.lax.bitcast_convert_type(xu & jnp.uint32(0xFFFF0000), jnp.float32)
    return lo, hi


def _router_body(x_ref, rw_ref, eb_ref, idx_ref, val_ref, xp_ref):
    x = x_ref[...]
    xp_ref[...] = _pack_bf16_pair(x[:, :D // 2], x[:, D // 2:])
    logits = jnp.dot(x, rw_ref[...], preferred_element_type=jnp.float32)
    logits = logits + eb_ref[...]
    sp = jnp.maximum(logits, 0.0) + jnp.log(1.0 + jnp.exp(-jnp.abs(logits)))
    act = jnp.sqrt(sp)
    m = x.shape[0]
    lane = lax.broadcasted_iota(jnp.int32, (m, LANES), 1)
    actm = jnp.where(lane < E, act, -1.0)
    m1 = jnp.max(actm, axis=1, keepdims=True)
    l1 = jnp.min(jnp.where(actm == m1, lane, LANES), axis=1, keepdims=True)
    act2 = jnp.where(lane == l1, -1.0, actm)
    m2 = jnp.max(act2, axis=1, keepdims=True)
    l2 = jnp.min(jnp.where(act2 == m2, lane, LANES), axis=1, keepdims=True)
    idx_ref[...] = jnp.concatenate([l1, l2], axis=1)
    val_ref[...] = jnp.concatenate([m1, m2], axis=1)


def _router(x2, routing_W, expert_bias):
    rw_pad = jnp.pad(routing_W, ((0, 0), (0, LANES - E)))
    eb_pad = jnp.pad(expert_bias, (0, LANES - E)).reshape(1, LANES)
    tm = 1024
    return pl.pallas_call(
        _router_body,
        grid=(N // tm,),
        in_specs=[
            pl.BlockSpec((tm, D), lambda t: (t, 0)),
            pl.BlockSpec((D, LANES), lambda t: (0, 0)),
            pl.BlockSpec((1, LANES), lambda t: (0, 0)),
        ],
        out_specs=[
            pl.BlockSpec((tm, 2), lambda t: (t, 0)),
            pl.BlockSpec((tm, 2), lambda t: (t, 0)),
            pl.BlockSpec((tm, D // 2), lambda t: (t, 0)),
        ],
        out_shape=[
            jax.ShapeDtypeStruct((N, 2), jnp.int32),
            jax.ShapeDtypeStruct((N, 2), jnp.float32),
            jax.ShapeDtypeStruct((N, D // 2), jnp.int32),
        ],
    )(x2, rw_pad, eb_pad)


def _dispatch(top_idx, top_vals):
    eid = top_idx.reshape(-1)
    oh = (eid[:, None] == jnp.arange(E, dtype=jnp.int32)[None, :]).astype(
        jnp.int32)
    ranks_inc = jnp.cumsum(oh, axis=0)
    cnt = ranks_inc[-1]
    rank = jnp.take_along_axis(ranks_inc, eid[:, None], axis=1)[:, 0] - 1
    padded = ((cnt + TM - 1) // TM) * TM
    pstart = jnp.concatenate([jnp.zeros(1, jnp.int32), jnp.cumsum(padded)])
    pos = pstart[eid] + rank
    gidx = jnp.zeros((P_CAP,), jnp.int32).at[pos].set(
        jnp.arange(NP, dtype=jnp.int32) // 2)
    wgt = jnp.zeros((P_CAP,), jnp.float32).at[pos].set(
        top_vals.reshape(-1)).reshape(P_CAP, 1)
    tstart = jnp.arange(NTILES, dtype=jnp.int32) * TM
    te = jnp.sum(tstart[:, None] >= pstart[1:][None, :], axis=1,
                 dtype=jnp.int32)
    te = jnp.minimum(te, E - 1)
    return gidx, wgt, te, pos


RPW = P_CAP // NW
CH = 32
NCH = RPW // CH
NB3 = 4


def _k3_body(x_hbm, gidx_hbm, xs_hbm,
             idx_b0, idx_b1, idx_b2, idx_b3,
             rows0, rows1, rows2, rows3,
             si0, si1, si2, si3, sg0, sg1, sg2, sg3, sw0, sw1, sw2, sw3):
    wid = lax.axis_index("s") * NC + lax.axis_index("c")
    base = wid * RPW
    idx_b = [idx_b0, idx_b1, idx_b2, idx_b3]
    rows = [rows0, rows1, rows2, rows3]
    sem_i = [si0, si1, si2, si3]
    sem_g = [sg0, sg1, sg2, sg3]
    sem_w = [sw0, sw1, sw2, sw3]

    def load_idx(c):
        b = c % NB3
        return pltpu.async_copy(gidx_hbm.at[pl.ds(base + c * CH, CH)],
                                idx_b[b], sem_i[b])

    def gather(c):
        b = c % NB3
        return pltpu.async_copy(x_hbm.at[idx_b[b]], rows[b], sem_g[b])

    def write(c):
        b = c % NB3
        return pltpu.async_copy(rows[b],
                                xs_hbm.at[pl.ds(base + c * CH, CH)],
                                sem_w[b])

    idn = [None] * NCH
    gd = [None] * NCH
    wd = [None] * NCH
    for c in range(NB3):
        idn[c] = load_idx(c)
    for c in range(NB3):
        idn[c].wait()
        gd[c] = gather(c)
    for c in range(NCH):
        gd[c].wait()
        wd[c] = write(c)
        n = c + NB3
        if n < NCH:
            idn[n] = load_idx(n)
            wd[c].wait()
            idn[n].wait()
            gd[n] = gather(n)
    for c in range(NCH - NB3, NCH):
        wd[c].wait()


def _sc_gather(x2, gidx):
    mesh = plsc.VectorSubcoreMesh(core_axis_name="c", subcore_axis_name="s")
    f = functools.partial(
        pl.kernel, mesh=mesh,
        out_type=jax.ShapeDtypeStruct((P_CAP, D // 2), jnp.int32),
        scratch_types=(
            [pltpu.VMEM((CH,), jnp.int32)] * 4
            + [pltpu.VMEM((CH, D // 2), jnp.int32)] * 4
            + [pltpu.SemaphoreType.DMA] * 12
        ),
    )(_k3_body)
    return f(x2, gidx)


def _group_body(te_ref, xs_ref, w1_ref, b1_ref, w2_ref, b2_ref, wgt_ref,
                y_ref):
    xlo, xhi = _unpack_bf16_pair(xs_ref[...])
    w1 = w1_ref[0]
    h = jnp.dot(xlo.astype(jnp.bfloat16),
                w1[:D // 2, :].astype(jnp.bfloat16),
                preferred_element_type=jnp.float32)
    h += jnp.dot(xhi.astype(jnp.bfloat16),
                 w1[D // 2:, :].astype(jnp.bfloat16),
                 preferred_element_type=jnp.float32)
    h += b1_ref[0]
    h = _silu(h)
    w2b = w2_ref[0].astype(jnp.bfloat16)
    y = jnp.dot(h.astype(jnp.bfloat16), w2b,
                preferred_element_type=jnp.float32) + b2_ref[0]
    y = y * wgt_ref[...]
    y_ref[...] = _pack_bf16_pair(y[:, :D // 2], y[:, D // 2:])


def _grouped_ffn(xs, routed_W1, routed_b1, routed_W2, routed_b2, wgt, te):
    b1 = routed_b1.reshape(E, 1, H)
    b2 = routed_b2.reshape(E, 1, D)
    grid_spec = pltpu.PrefetchScalarGridSpec(
        num_scalar_prefetch=1,
        grid=(NTILES,),
        in_specs=[
            pl.BlockSpec((TM, D // 2), lambda t, te: (t, 0)),
            pl.BlockSpec((1, D, H), lambda t, te: (te[t], 0, 0)),
            pl.BlockSpec((1, 1, H), lambda t, te: (te[t], 0, 0)),
            pl.BlockSpec((1, H, D), lambda t, te: (te[t], 0, 0)),
            pl.BlockSpec((1, 1, D), lambda t, te: (te[t], 0, 0)),
            pl.BlockSpec((TM, 1), lambda t, te: (t, 0)),
        ],
        out_specs=pl.BlockSpec((TM, D // 2), lambda t, te: (t, 0)),
    )
    return pl.pallas_call(
        _group_body,
        grid_spec=grid_spec,
        out_shape=jax.ShapeDtypeStruct((P_CAP, D // 2), jnp.int32),
        compiler_params=pltpu.CompilerParams(
            dimension_semantics=("arbitrary",),
        ),
    )(te, xs, routed_W1, b1, routed_W2, b2, wgt)


def _shared_body(x_ref, w1_ref, b1_ref, w2_ref, b2_ref, s_ref):
    x = x_ref[...]
    h = jnp.dot(x, w1_ref[...], preferred_element_type=jnp.float32) + b1_ref[...]
    h = _silu(h)
    s_ref[...] = jnp.dot(h, w2_ref[...], preferred_element_type=jnp.float32) \
        + b2_ref[...]


def _shared_ffn(x2, W1, b1, W2, b2):
    tm = 512
    return pl.pallas_call(
        _shared_body,
        grid=(N // tm,),
        in_specs=[
            pl.BlockSpec((tm, D), lambda t: (t, 0)),
            pl.BlockSpec((D, H), lambda t: (0, 0)),
            pl.BlockSpec((1, H), lambda t: (0, 0)),
            pl.BlockSpec((H, D), lambda t: (0, 0)),
            pl.BlockSpec((1, D), lambda t: (0, 0)),
        ],
        out_specs=pl.BlockSpec((tm, D), lambda t: (t, 0)),
        out_shape=jax.ShapeDtypeStruct((N, D), jnp.float32),
    )(x2, W1, b1.reshape(1, H), W2, b2.reshape(1, D))


TPW = N // NW
CC = 16
NCC = TPW // CC


def _k6_body(s_hbm, y_hbm, pos_hbm, out_hbm,
             pidx_b0, pidx_b1, y01_0, y01_1, s_0, s_1,
             sem_i0, sem_i1, sem_g0, sem_g1, sem_s0, sem_s1,
             sem_w0, sem_w1):
    wid = lax.axis_index("s") * NC + lax.axis_index("c")
    base = wid * TPW
    pidx_b = [pidx_b0, pidx_b1]
    y01 = [y01_0, y01_1]
    sb = [s_0, s_1]
    sem_i = [sem_i0, sem_i1]
    sem_g = [sem_g0, sem_g1]
    sem_s = [sem_s0, sem_s1]
    sem_w = [sem_w0, sem_w1]

    def load_pidx(c):
        b = c % 2
        return pltpu.async_copy(pos_hbm.at[pl.ds(2 * (base + c * CC), 2 * CC)],
                                pidx_b[b], sem_i[b])

    gd = [None] * NCC
    sd = [None] * NCC
    wd = [None] * NCC
    idn = [load_pidx(0), load_pidx(1)]
    idn[0].wait()
    gd[0] = pltpu.async_copy(y_hbm.at[pidx_b[0]], y01[0], sem_g[0])
    sd[0] = pltpu.async_copy(s_hbm.at[pl.ds(base, CC)], sb[0], sem_s[0])
    for c in range(NCC):
        b = c % 2
        if c + 1 < NCC:
            nb = (c + 1) % 2
            idn[nb].wait()
            if c >= 1:
                wd[c - 1].wait()
            gd[c + 1] = pltpu.async_copy(y_hbm.at[pidx_b[nb]], y01[nb],
                                         sem_g[nb])
            sd[c + 1] = pltpu.async_copy(
                s_hbm.at[pl.ds(base + (c + 1) * CC, CC)], sb[nb], sem_s[nb])
        gd[c].wait()
        if c + 2 < NCC:
            idn[b] = load_pidx(c + 2)
        sd[c].wait()

        def row(r, _):
            def col(i, _):
                for u in range(2):
                    g = i * 2 + u
                    cs = pl.ds(g * 16, 16)
                    csh = pl.ds(D // 2 + g * 16, 16)
                    lo0, hi0 = _unpack_bf16_pair(y01[b][2 * r, cs])
                    lo1, hi1 = _unpack_bf16_pair(y01[b][2 * r + 1, cs])
                    sb[b][r, cs] = sb[b][r, cs] + (lo0 + lo1)
                    sb[b][r, csh] = sb[b][r, csh] + (hi0 + hi1)
                return 0
            lax.fori_loop(0, D // 64, col, 0)
            return 0

        lax.fori_loop(0, CC, row, 0)
        wd[c] = pltpu.async_copy(sb[b], out_hbm.at[pl.ds(base + c * CC, CC)],
                                 sem_w[b])
    wd[NCC - 2].wait()
    wd[NCC - 1].wait()


def _sc_combine(S, Y, pos):
    mesh = plsc.VectorSubcoreMesh(core_axis_name="c", subcore_axis_name="s")
    f = functools.partial(
        pl.kernel, mesh=mesh,
        out_type=jax.ShapeDtypeStruct((N, D), jnp.float32),
        scratch_types=[
            pltpu.VMEM((2 * CC,), jnp.int32),
            pltpu.VMEM((2 * CC,), jnp.int32),
            pltpu.VMEM((2 * CC, D // 2), jnp.int32),
            pltpu.VMEM((2 * CC, D // 2), jnp.int32),
            pltpu.VMEM((CC, D), jnp.float32),
            pltpu.VMEM((CC, D), jnp.float32),
            pltpu.SemaphoreType.DMA,
            pltpu.SemaphoreType.DMA,
            pltpu.SemaphoreType.DMA,
            pltpu.SemaphoreType.DMA,
            pltpu.SemaphoreType.DMA,
            pltpu.SemaphoreType.DMA,
            pltpu.SemaphoreType.DMA,
            pltpu.SemaphoreType.DMA,
        ],
    )(_k6_body)
    return f(S, Y, pos)


def kernel(X, shared_W1, shared_b1, shared_W2, shared_b2,
           routed_W1, routed_b1, routed_W2, routed_b2,
           routing_W, expert_bias):
    B, T, _ = X.shape
    x2 = X.reshape(N, D)

    top_idx, top_vals, xp = _router(x2, routing_W, expert_bias)
    gidx, wgt, te, pos = _dispatch(top_idx, top_vals)
    xs = _sc_gather(xp, gidx)
    y = _grouped_ffn(xs, routed_W1, routed_b1, routed_W2, routed_b2, wgt, te)
    s = _shared_ffn(x2, shared_W1, shared_b1, shared_W2, shared_b2)
    out = _sc_combine(s, y, pos)
    return out.reshape(B, T, D)


# speedup vs baseline: 1.1954x; 1.0011x over previous
import functools
import jax
import jax.numpy as jnp
from jax import lax
from jax.experimental import pallas as pl
from jax.experimental.pallas import tpu as pltpu
from jax.experimental.pallas import tpu_sc as plsc

D = 1024
E = 8
H = 512
N = 4096
NP = 2 * N
TM = 256
P_CAP = NP + E * TM
NTILES = P_CAP // TM
LANES = 128

NC, NS = 2, 16
NW = NC * NS


def _silu(x):
    return x / (1.0 + jnp.exp(-x))


def _pack_bf16_pair(a, b):
    au = jax.lax.bitcast_convert_type(a, jnp.uint32) >> 16
    bu = jax.lax.bitcast_convert_type(b, jnp.uint32) & jnp.uint32(0xFFFF0000)
    return jax.lax.bitcast_convert_type(au | bu, jnp.int32)


def _unpack_bf16_pair(xi):
    xu = jax.lax.bitcast_convert_type(xi, jnp.uint32)
    lo = jax.lax.bitcast_convert_type(xu << 16, jnp.float32)
    hi = jax.lax.bitcast_convert_type(xu & jnp.uint32(0xFFFF0000), jnp.float32)
    return lo, hi


def _router_body(x_ref, rw_ref, eb_ref, idx_ref, val_ref, xp_ref):
    x = x_ref[...]
    xp_ref[...] = _pack_bf16_pair(x[:, :D // 2], x[:, D // 2:])
    logits = jnp.dot(x, rw_ref[...], preferred_element_type=jnp.float32)
    logits = logits + eb_ref[...]
    sp = jnp.maximum(logits, 0.0) + jnp.log(1.0 + jnp.exp(-jnp.abs(logits)))
    act = jnp.sqrt(sp)
    m = x.shape[0]
    lane = lax.broadcasted_iota(jnp.int32, (m, LANES), 1)
    actm = jnp.where(lane < E, act, -1.0)
    m1 = jnp.max(actm, axis=1, keepdims=True)
    l1 = jnp.min(jnp.where(actm == m1, lane, LANES), axis=1, keepdims=True)
    act2 = jnp.where(lane == l1, -1.0, actm)
    m2 = jnp.max(act2, axis=1, keepdims=True)
    l2 = jnp.min(jnp.where(act2 == m2, lane, LANES), axis=1, keepdims=True)
    idx_ref[...] = jnp.concatenate([l1, l2], axis=1)
    val_ref[...] = jnp.concatenate([m1, m2], axis=1)


def _router(x2, routing_W, expert_bias):
    rw_pad = jnp.pad(routing_W, ((0, 0), (0, LANES - E)))
    eb_pad = jnp.pad(expert_bias, (0, LANES - E)).reshape(1, LANES)
    tm = 1024
    return pl.pallas_call(
        _router_body,
        grid=(N // tm,),
        in_specs=[
            pl.BlockSpec((tm, D), lambda t: (t, 0)),
            pl.BlockSpec((D, LANES), lambda t: (0, 0)),
            pl.BlockSpec((1, LANES), lambda t: (0, 0)),
        ],
        out_specs=[
            pl.BlockSpec((tm, 2), lambda t: (t, 0)),
            pl.BlockSpec((tm, 2), lambda t: (t, 0)),
            pl.BlockSpec((tm, D // 2), lambda t: (t, 0)),
        ],
        out_shape=[
            jax.ShapeDtypeStruct((N, 2), jnp.int32),
            jax.ShapeDtypeStruct((N, 2), jnp.float32),
            jax.ShapeDtypeStruct((N, D // 2), jnp.int32),
        ],
    )(x2, rw_pad, eb_pad)


def _dispatch(top_idx, top_vals):
    eid = top_idx.reshape(-1)
    oh = (eid[:, None] == jnp.arange(E, dtype=jnp.int32)[None, :]).astype(
        jnp.int32)
    ranks_inc = jnp.cumsum(oh, axis=0)
    cnt = ranks_inc[-1]
    rank = jnp.take_along_axis(ranks_inc, eid[:, None], axis=1)[:, 0] - 1
    padded = ((cnt + TM - 1) // TM) * TM
    pstart = jnp.concatenate([jnp.zeros(1, jnp.int32), jnp.cumsum(padded)])
    pos = pstart[eid] + rank
    gidx = jnp.zeros((P_CAP,), jnp.int32).at[pos].set(
        jnp.arange(NP, dtype=jnp.int32) // 2)
    wgt = jnp.zeros((P_CAP,), jnp.float32).at[pos].set(
        top_vals.reshape(-1)).reshape(P_CAP, 1)
    tstart = jnp.arange(NTILES, dtype=jnp.int32) * TM
    te = jnp.sum(tstart[:, None] >= pstart[1:][None, :], axis=1,
                 dtype=jnp.int32)
    te = jnp.minimum(te, E - 1)
    return gidx, wgt, te, pos


RPW = P_CAP // NW
CH = 32
NCH = RPW // CH
NB3 = 6


def _k3_body(x_hbm, gidx_hbm, xs_hbm, *refs):
    wid = lax.axis_index("s") * NC + lax.axis_index("c")
    base = wid * RPW
    idx_b = refs[0:NB3]
    rows = refs[NB3:2 * NB3]
    sem_i = refs[2 * NB3:3 * NB3]
    sem_g = refs[3 * NB3:4 * NB3]
    sem_w = refs[4 * NB3:5 * NB3]

    def load_idx(c):
        b = c % NB3
        return pltpu.async_copy(gidx_hbm.at[pl.ds(base + c * CH, CH)],
                                idx_b[b], sem_i[b])

    def gather(c):
        b = c % NB3
        return pltpu.async_copy(x_hbm.at[idx_b[b]], rows[b], sem_g[b])

    def write(c):
        b = c % NB3
        return pltpu.async_copy(rows[b],
                                xs_hbm.at[pl.ds(base + c * CH, CH)],
                                sem_w[b])

    idn = [None] * NCH
    gd = [None] * NCH
    wd = [None] * NCH
    npro = min(NB3, NCH)
    for c in range(npro):
        idn[c] = load_idx(c)
    for c in range(npro):
        idn[c].wait()
        gd[c] = gather(c)
    for c in range(NCH):
        gd[c].wait()
        wd[c] = write(c)
        m = c + NB3 - 1
        if NB3 <= m < NCH:
            idn[m].wait()
            wd[m - NB3].wait()
            gd[m] = gather(m)
        n = c + NB3
        if n < NCH:
            idn[n] = load_idx(n)
    for c in range(max(0, NCH - NB3), NCH):
        wd[c].wait()


def _sc_gather(x2, gidx):
    mesh = plsc.VectorSubcoreMesh(core_axis_name="c", subcore_axis_name="s")
    f = functools.partial(
        pl.kernel, mesh=mesh,
        out_type=jax.ShapeDtypeStruct((P_CAP, D // 2), jnp.int32),
        scratch_types=(
            [pltpu.VMEM((CH,), jnp.int32)] * NB3
            + [pltpu.VMEM((CH, D // 2), jnp.int32)] * NB3
            + [pltpu.SemaphoreType.DMA] * (3 * NB3)
        ),
    )(_k3_body)
    return f(x2, gidx)


def _group_body(te_ref, xs_ref, w1_ref, b1_ref, w2_ref, b2_ref, wgt_ref,
                y_ref):
    xlo, xhi = _unpack_bf16_pair(xs_ref[...])
    w1 = w1_ref[0]
    h = jnp.dot(xlo.astype(jnp.bfloat16),
                w1[:D // 2, :].astype(jnp.bfloat16),
                preferred_element_type=jnp.float32)
    h += jnp.dot(xhi.astype(jnp.bfloat16),
                 w1[D // 2:, :].astype(jnp.bfloat16),
                 preferred_element_type=jnp.float32)
    h += b1_ref[0]
    h = _silu(h)
    w2b = w2_ref[0].astype(jnp.bfloat16)
    y = jnp.dot(h.astype(jnp.bfloat16), w2b,
                preferred_element_type=jnp.float32) + b2_ref[0]
    y = y * wgt_ref[...]
    y_ref[...] = _pack_bf16_pair(y[:, :D // 2], y[:, D // 2:])


def _grouped_ffn(xs, routed_W1, routed_b1, routed_W2, routed_b2, wgt, te):
    b1 = routed_b1.reshape(E, 1, H)
    b2 = routed_b2.reshape(E, 1, D)
    grid_spec = pltpu.PrefetchScalarGridSpec(
        num_scalar_prefetch=1,
        grid=(NTILES,),
        in_specs=[
            pl.BlockSpec((TM, D // 2), lambda t, te: (t, 0)),
            pl.BlockSpec((1, D, H), lambda t, te: (te[t], 0, 0)),
            pl.BlockSpec((1, 1, H), lambda t, te: (te[t], 0, 0)),
            pl.BlockSpec((1, H, D), lambda t, te: (te[t], 0, 0)),
            pl.BlockSpec((1, 1, D), lambda t, te: (te[t], 0, 0)),
            pl.BlockSpec((TM, 1), lambda t, te: (t, 0)),
        ],
        out_specs=pl.BlockSpec((TM, D // 2), lambda t, te: (t, 0)),
    )
    return pl.pallas_call(
        _group_body,
        grid_spec=grid_spec,
        out_shape=jax.ShapeDtypeStruct((P_CAP, D // 2), jnp.int32),
        compiler_params=pltpu.CompilerParams(
            dimension_semantics=("arbitrary",),
        ),
    )(te, xs, routed_W1, b1, routed_W2, b2, wgt)


def _shared_body(x_ref, w1_ref, b1_ref, w2_ref, b2_ref, s_ref):
    x = x_ref[...]
    h = jnp.dot(x, w1_ref[...], preferred_element_type=jnp.float32) + b1_ref[...]
    h = _silu(h)
    s_ref[...] = jnp.dot(h, w2_ref[...], preferred_element_type=jnp.float32) \
        + b2_ref[...]


def _shared_ffn(x2, W1, b1, W2, b2):
    tm = 512
    return pl.pallas_call(
        _shared_body,
        grid=(N // tm,),
        in_specs=[
            pl.BlockSpec((tm, D), lambda t: (t, 0)),
            pl.BlockSpec((D, H), lambda t: (0, 0)),
            pl.BlockSpec((1, H), lambda t: (0, 0)),
            pl.BlockSpec((H, D), lambda t: (0, 0)),
            pl.BlockSpec((1, D), lambda t: (0, 0)),
        ],
        out_specs=pl.BlockSpec((tm, D), lambda t: (t, 0)),
        out_shape=jax.ShapeDtypeStruct((N, D), jnp.float32),
    )(x2, W1, b1.reshape(1, H), W2, b2.reshape(1, D))


TPW = N // NW
CC = 16
NCC = TPW // CC


def _k6_body(s_hbm, y_hbm, pos_hbm, out_hbm,
             pidx_b0, pidx_b1, y01_0, y01_1, s_0, s_1,
             sem_i0, sem_i1, sem_g0, sem_g1, sem_s0, sem_s1,
             sem_w0, sem_w1):
    wid = lax.axis_index("s") * NC + lax.axis_index("c")
    base = wid * TPW
    pidx_b = [pidx_b0, pidx_b1]
    y01 = [y01_0, y01_1]
    sb = [s_0, s_1]
    sem_i = [sem_i0, sem_i1]
    sem_g = [sem_g0, sem_g1]
    sem_s = [sem_s0, sem_s1]
    sem_w = [sem_w0, sem_w1]

    def load_pidx(c):
        b = c % 2
        return pltpu.async_copy(pos_hbm.at[pl.ds(2 * (base + c * CC), 2 * CC)],
                                pidx_b[b], sem_i[b])

    gd = [None] * NCC
    sd = [None] * NCC
    wd = [None] * NCC
    idn = [load_pidx(0), load_pidx(1)]
    idn[0].wait()
    gd[0] = pltpu.async_copy(y_hbm.at[pidx_b[0]], y01[0], sem_g[0])
    sd[0] = pltpu.async_copy(s_hbm.at[pl.ds(base, CC)], sb[0], sem_s[0])
    for c in range(NCC):
        b = c % 2
        if c + 1 < NCC:
            nb = (c + 1) % 2
            idn[nb].wait()
            if c >= 1:
                wd[c - 1].wait()
            gd[c + 1] = pltpu.async_copy(y_hbm.at[pidx_b[nb]], y01[nb],
                                         sem_g[nb])
            sd[c + 1] = pltpu.async_copy(
                s_hbm.at[pl.ds(base + (c + 1) * CC, CC)], sb[nb], sem_s[nb])
        gd[c].wait()
        if c + 2 < NCC:
            idn[b] = load_pidx(c + 2)
        sd[c].wait()

        def row(r, _):
            def col(i, _):
                for u in range(2):
                    g = i * 2 + u
                    cs = pl.ds(g * 16, 16)
                    csh = pl.ds(D // 2 + g * 16, 16)
                    lo0, hi0 = _unpack_bf16_pair(y01[b][2 * r, cs])
                    lo1, hi1 = _unpack_bf16_pair(y01[b][2 * r + 1, cs])
                    sb[b][r, cs] = sb[b][r, cs] + (lo0 + lo1)
                    sb[b][r, csh] = sb[b][r, csh] + (hi0 + hi1)
                return 0
            lax.fori_loop(0, D // 64, col, 0)
            return 0

        lax.fori_loop(0, CC, row, 0)
        wd[c] = pltpu.async_copy(sb[b], out_hbm.at[pl.ds(base + c * CC, CC)],
                                 sem_w[b])
    wd[NCC - 2].wait()
    wd[NCC - 1].wait()


def _sc_combine(S, Y, pos):
    mesh = plsc.VectorSubcoreMesh(core_axis_name="c", subcore_axis_name="s")
    f = functools.partial(
        pl.kernel, mesh=mesh,
        out_type=jax.ShapeDtypeStruct((N, D), jnp.float32),
        scratch_types=[
            pltpu.VMEM((2 * CC,), jnp.int32),
            pltpu.VMEM((2 * CC,), jnp.int32),
            pltpu.VMEM((2 * CC, D // 2), jnp.int32),
            pltpu.VMEM((2 * CC, D // 2), jnp.int32),
            pltpu.VMEM((CC, D), jnp.float32),
            pltpu.VMEM((CC, D), jnp.float32),
            pltpu.SemaphoreType.DMA,
            pltpu.SemaphoreType.DMA,
            pltpu.SemaphoreType.DMA,
            pltpu.SemaphoreType.DMA,
            pltpu.SemaphoreType.DMA,
            pltpu.SemaphoreType.DMA,
            pltpu.SemaphoreType.DMA,
            pltpu.SemaphoreType.DMA,
        ],
    )(_k6_body)
    return f(S, Y, pos)


def kernel(X, shared_W1, shared_b1, shared_W2, shared_b2,
           routed_W1, routed_b1, routed_W2, routed_b2,
           routing_W, expert_bias):
    B, T, _ = X.shape
    x2 = X.reshape(N, D)

    top_idx, top_vals, xp = _router(x2, routing_W, expert_bias)
    gidx, wgt, te, pos = _dispatch(top_idx, top_vals)
    xs = _sc_gather(xp, gidx)
    y = _grouped_ffn(xs, routed_W1, routed_b1, routed_W2, routed_b2, wgt, te)
    s = _shared_ffn(x2, shared_W1, shared_b1, shared_W2, shared_b2)
    out = _sc_combine(s, y, pos)
    return out.reshape(B, T, D)
